# baseline (speedup 1.0000x reference)
import jax
import jax.numpy as jnp
from jax.experimental import pallas as pl
from jax.experimental.pallas import tpu as pltpu

_RATIO = 0.2


def _cut_bounds(key, h, w):
    cut_x = int(w * _RATIO + 0.5)
    cut_y = int(h * _RATIO + 0.5)
    k1, k2 = jax.random.split(key)
    offset_x = jax.random.randint(k1, (1, 1), 0, w + (1 - cut_x % 2))[0, 0]
    offset_y = jax.random.randint(k2, (1, 1), 0, h + (1 - cut_y % 2))[0, 0]
    x0 = jnp.clip(offset_x - cut_x // 2, 0, w - 1)
    x1 = jnp.clip(offset_x - cut_x // 2 + cut_x - 1, 0, w - 1)
    y0 = jnp.clip(offset_y - cut_y // 2, 0, h - 1)
    y1 = jnp.clip(offset_y - cut_y // 2 + cut_y - 1, 0, h - 1)
    return jnp.stack([y0, y1, x0, x1]).astype(jnp.int32)


def _body(b_ref, x_ref, o_ref):
    by, w, c = x_ref.shape
    i = pl.program_id(0)
    r0 = i * by
    y0, y1, x0, x1 = b_ref[0], b_ref[1], b_ref[2], b_ref[3]
    intersects = (r0 <= y1) & (r0 + by - 1 >= y0)

    @pl.when(intersects)
    def _masked():
        rows = r0 + jax.lax.broadcasted_iota(jnp.int32, (by, w, c), 0)
        cols = jax.lax.broadcasted_iota(jnp.int32, (by, w, c), 1)
        inside = (rows >= y0) & (rows <= y1) & (cols >= x0) & (cols <= x1)
        o_ref[...] = jnp.where(inside, jnp.zeros_like(o_ref), x_ref[...])

    @pl.when(jnp.logical_not(intersects))
    def _copy():
        o_ref[...] = x_ref[...]


def kernel(x, key):
    h, w, c = x.shape
    bounds = _cut_bounds(key, h, w)
    by = 16
    return pl.pallas_call(
        _body,
        grid=(h // by,),
        in_specs=[
            pl.BlockSpec(memory_space=pltpu.SMEM),
            pl.BlockSpec((by, w, c), lambda i: (i, 0, 0)),
        ],
        out_specs=pl.BlockSpec((by, w, c), lambda i: (i, 0, 0)),
        out_shape=jax.ShapeDtypeStruct((h, w, c), x.dtype),
    )(bounds, x)


# speedup vs baseline: 1.1905x; 1.1905x over previous
import jax
import jax.numpy as jnp
from jax.experimental import pallas as pl
from jax.experimental.pallas import tpu as pltpu

_RATIO = 0.2


def _tf2x32(k0, k1, c0, c1):
    u = jnp.uint32
    ks2 = k0 ^ k1 ^ u(0x1BD11BDA)
    ks = (k0, k1, ks2)
    x0 = c0 + k0
    x1 = c1 + k1
    rots = ((13, 15, 26, 6), (17, 29, 16, 24))
    for i in range(5):
        for r in rots[i % 2]:
            x0 = x0 + x1
            x1 = ((x1 << u(r)) | (x1 >> u(32 - r))) ^ x0
        x0 = x0 + ks[(i + 1) % 3]
        x1 = x1 + ks[(i + 2) % 3] + u(i + 1)
    return x0, x1


def _randint_mod(k0, k1, span):
    u = jnp.uint32
    a0, a1 = _tf2x32(k0, k1, u(0), u(0))
    b0, b1 = _tf2x32(k0, k1, u(0), u(1))
    h0, h1 = _tf2x32(a0, a1, u(0), u(0))
    l0, l1 = _tf2x32(b0, b1, u(0), u(0))
    higher = h0 ^ h1
    lower = l0 ^ l1
    mult = ((2 ** 16 % span) ** 2) % span
    off = ((higher % u(span)) * u(mult) + (lower % u(span))) % u(span)
    return off.astype(jnp.int32)


def _window(key_ref, h, w):
    u = jnp.uint32
    cut_x = int(w * _RATIO + 0.5)
    cut_y = int(h * _RATIO + 0.5)
    k0, k1 = key_ref[0], key_ref[1]
    ka0, ka1 = _tf2x32(k0, k1, u(0), u(0))
    kb0, kb1 = _tf2x32(k0, k1, u(0), u(1))
    ox = _randint_mod(ka0, ka1, w + (1 - cut_x % 2))
    oy = _randint_mod(kb0, kb1, h + (1 - cut_y % 2))
    x0 = jnp.maximum(ox - cut_x // 2, 0)
    x1 = jnp.minimum(ox - cut_x // 2 + cut_x - 1, w - 1)
    y0 = jnp.maximum(oy - cut_y // 2, 0)
    y1 = jnp.minimum(oy - cut_y // 2 + cut_y - 1, h - 1)
    return y0, y1, x0, x1


def _body(key_ref, x_ref, o_ref):
    by, w, c = x_ref.shape
    h = pl.num_programs(0) * by
    i = pl.program_id(0)
    r0 = i * by
    y0, y1, x0, x1 = _window(key_ref, h, w)
    intersects = (r0 <= y1) & (r0 + by - 1 >= y0)

    @pl.when(intersects)
    def _masked():
        rows = r0 + jax.lax.broadcasted_iota(jnp.int32, (by, w, c), 0)
        cols = jax.lax.broadcasted_iota(jnp.int32, (by, w, c), 1)
        inside = (rows >= y0) & (rows <= y1) & (cols >= x0) & (cols <= x1)
        o_ref[...] = jnp.where(inside, jnp.zeros_like(o_ref), x_ref[...])

    @pl.when(jnp.logical_not(intersects))
    def _copy():
        o_ref[...] = x_ref[...]


def kernel(x, key):
    h, w, c = x.shape
    key_raw = jax.random.key_data(key).astype(jnp.uint32)
    by = 16
    return pl.pallas_call(
        _body,
        grid=(h // by,),
        in_specs=[
            pl.BlockSpec(memory_space=pltpu.SMEM),
            pl.BlockSpec((by, w, c), lambda i: (i, 0, 0)),
        ],
        out_specs=pl.BlockSpec((by, w, c), lambda i: (i, 0, 0)),
        out_shape=jax.ShapeDtypeStruct((h, w, c), x.dtype),
    )(key_raw, x)
